# baseline (speedup 1.0000x reference)
import functools

import jax
import jax.numpy as jnp
from jax import lax
from jax.experimental import pallas as pl
from jax.experimental.pallas import tpu as pltpu
from jax.experimental.pallas import tpu_sc as plsc

N_USERS = 100000
N_ITEMS = 100000
D = 64
B = 16384

NC, NS, L = 2, 16, 16
NW = NC * NS
BPW = B // NW
CHUNK = 128
NCHUNK = BPW // CHUNK

_mesh = plsc.VectorSubcoreMesh(core_axis_name="c", subcore_axis_name="s")


@functools.partial(
    pl.kernel,
    out_type=jax.ShapeDtypeStruct((B,), jnp.float32),
    mesh=_mesh,
    compiler_params=pltpu.CompilerParams(
        needs_layout_passes=False, use_tc_tiling_on_sc=False),
    scratch_types=[
        pltpu.VMEM((NCHUNK, CHUNK), jnp.int32),
        pltpu.VMEM((NCHUNK, CHUNK), jnp.int32),
        pltpu.VMEM((BPW, D), jnp.float32),
        pltpu.VMEM((BPW, D), jnp.float32),
        pltpu.VMEM((BPW,), jnp.float32),
        pltpu.VMEM((BPW,), jnp.float32),
        pltpu.VMEM((BPW,), jnp.float32),
        pltpu.SemaphoreType.DMA,
    ],
)
def _mf_sc(uids, iids, uemb, iemb, ubias, ibias, out,
           uidx, iidx, urows, irows, ub_v, ib_v, out_v, sem):
    wid = lax.axis_index("s") * NC + lax.axis_index("c")
    base = wid * BPW
    r0 = wid * NCHUNK

    pltpu.sync_copy(uids.at[pl.ds(r0, NCHUNK)], uidx)
    pltpu.sync_copy(iids.at[pl.ds(r0, NCHUNK)], iidx)

    cps = []
    for j in range(NCHUNK):
        cps.append(pltpu.async_copy(
            uemb.at[uidx.at[j]], urows.at[pl.ds(j * CHUNK, CHUNK)], sem))
        cps.append(pltpu.async_copy(
            iemb.at[iidx.at[j]], irows.at[pl.ds(j * CHUNK, CHUNK)], sem))
        cps.append(pltpu.async_copy(
            ubias.at[uidx.at[j]], ub_v.at[pl.ds(j * CHUNK, CHUNK)], sem))
        cps.append(pltpu.async_copy(
            ibias.at[iidx.at[j]], ib_v.at[pl.ds(j * CHUNK, CHUNK)], sem))
    for c in cps:
        c.wait()

    lanes = lax.broadcasted_iota(jnp.int32, (L,), 0)

    def body(g, carry):
        b0 = g * L
        rows = b0 + lanes
        acc = ub_v[pl.ds(b0, L)] + ib_v[pl.ds(b0, L)]
        for d in range(D):
            dv = jnp.full((L,), d, jnp.int32)
            acc = acc + (plsc.load_gather(urows, [rows, dv])
                         * plsc.load_gather(irows, [rows, dv]))
        out_v[pl.ds(b0, L)] = acc
        return carry

    lax.fori_loop(0, BPW // L, body, 0)

    pltpu.sync_copy(out_v, out.at[pl.ds(base, BPW)])


def kernel(user_ids, item_ids, user_embedding, item_embedding,
           user_bias, item_bias, global_bias):
    uids = user_ids.astype(jnp.int32).reshape(B // CHUNK, CHUNK)
    iids = item_ids.astype(jnp.int32).reshape(B // CHUNK, CHUNK)
    out = _mf_sc(uids, iids, user_embedding, item_embedding,
                 user_bias.reshape(-1), item_bias.reshape(-1))
    return out[:, None] + global_bias
